# baseline (device time: 77879 ns/iter reference)
import jax
import jax.numpy as jnp
from jax import lax
from jax.experimental import pallas as pl
from jax.experimental.pallas import tpu as pltpu

N_DEV = 16


def kernel(x, w_mat):
    m_per, k = x.shape
    _, n_per = w_mat.shape

    def body(x_ref, w_ref, out_ref, comm_ref, send_sems, recv_sems):
        my_pos = lax.axis_index("i")
        left = lax.rem(my_pos + N_DEV - 1, N_DEV)
        right = lax.rem(my_pos + 1, N_DEV)

        barrier_sem = pltpu.get_barrier_semaphore()
        for nbr in (left, right):
            pl.semaphore_signal(
                barrier_sem, inc=1,
                device_id=(nbr,), device_id_type=pl.DeviceIdType.MESH,
            )
        pl.semaphore_wait(barrier_sem, 2)

        comm_ref[0] = x_ref[...]
        out_ref[pl.ds(my_pos * m_per, m_per), :] = jnp.dot(
            x_ref[...], w_ref[...], preferred_element_type=jnp.float32
        )

        for h in range(N_DEV - 1):
            rdma = pltpu.make_async_remote_copy(
                src_ref=comm_ref.at[h],
                dst_ref=comm_ref.at[h + 1],
                send_sem=send_sems.at[h],
                recv_sem=recv_sems.at[h],
                device_id=(right,),
                device_id_type=pl.DeviceIdType.MESH,
            )
            rdma.start()
            rdma.wait()

            origin = lax.rem(my_pos + N_DEV - (h + 1), N_DEV)
            out_ref[pl.ds(origin * m_per, m_per), :] = jnp.dot(
                comm_ref[h + 1], w_ref[...],
                preferred_element_type=jnp.float32,
            )

    return pl.pallas_call(
        body,
        out_shape=jax.ShapeDtypeStruct((N_DEV * m_per, n_per), jnp.float32),
        in_specs=[
            pl.BlockSpec(memory_space=pltpu.VMEM),
            pl.BlockSpec(memory_space=pltpu.VMEM),
        ],
        out_specs=pl.BlockSpec(memory_space=pltpu.VMEM),
        scratch_shapes=[
            pltpu.VMEM((N_DEV, m_per, k), jnp.float32),
            pltpu.SemaphoreType.DMA((N_DEV - 1,)),
            pltpu.SemaphoreType.DMA((N_DEV - 1,)),
        ],
        compiler_params=pltpu.CompilerParams(collective_id=0),
    )(x, w_mat)


# device time: 43708 ns/iter; 1.7818x vs baseline; 1.7818x over previous
import jax
import jax.numpy as jnp
from jax import lax
from jax.experimental import pallas as pl
from jax.experimental.pallas import tpu as pltpu

N_DEV = 16
H_R = 8
H_L = 7


def kernel(x, w_mat):
    m_per, k = x.shape
    _, n_per = w_mat.shape

    def ls(d):
        return 0 if d == 0 else H_R + d

    def body(x_ref, w_ref, out_ref, comm_ref,
             send_r, recv_r, send_l, recv_l):
        my_pos = lax.axis_index("i")
        left = lax.rem(my_pos + N_DEV - 1, N_DEV)
        right = lax.rem(my_pos + 1, N_DEV)

        barrier_sem = pltpu.get_barrier_semaphore()
        for nbr in (left, right):
            pl.semaphore_signal(
                barrier_sem, inc=1,
                device_id=(nbr,), device_id_type=pl.DeviceIdType.MESH,
            )
        pl.semaphore_wait(barrier_sem, 2)

        comm_ref[0] = x_ref[...]

        def send(slot_src, slot_dst, sems, h, target):
            rdma = pltpu.make_async_remote_copy(
                src_ref=comm_ref.at[slot_src],
                dst_ref=comm_ref.at[slot_dst],
                send_sem=sems.at[h],
                recv_sem=(recv_r if sems is send_r else recv_l).at[h],
                device_id=(target,),
                device_id_type=pl.DeviceIdType.MESH,
            )
            rdma.start()
            return rdma

        def recv(slot_dst, sems, h, src_dev):
            return pltpu.make_async_remote_copy(
                src_ref=comm_ref.at[slot_dst],
                dst_ref=comm_ref.at[slot_dst],
                send_sem=sems.at[h],
                recv_sem=sems.at[h],
                device_id=(src_dev,),
                device_id_type=pl.DeviceIdType.MESH,
            )

        def gemm(slot, origin):
            out_ref[pl.ds(origin * m_per, m_per), :] = jnp.dot(
                comm_ref[slot], w_ref[...],
                preferred_element_type=jnp.float32,
            )

        sends = []
        sends.append(send(0, 1, send_r, 0, right))
        sends.append(send(0, ls(1), send_l, 0, left))
        gemm(0, my_pos)

        for h in range(H_R):
            recv(h + 1, recv_r, h, left).wait_recv()
            if h + 1 < H_R:
                sends.append(send(h + 1, h + 2, send_r, h + 1, right))
            gemm(h + 1, lax.rem(my_pos + N_DEV - (h + 1), N_DEV))

            if h < H_L:
                recv(ls(h + 1), recv_l, h, right).wait_recv()
                if h + 1 < H_L:
                    sends.append(
                        send(ls(h + 1), ls(h + 2), send_l, h + 1, left)
                    )
                gemm(ls(h + 1), lax.rem(my_pos + h + 1, N_DEV))

        for s in sends:
            s.wait_send()

    return pl.pallas_call(
        body,
        out_shape=jax.ShapeDtypeStruct((N_DEV * m_per, n_per), jnp.float32),
        in_specs=[
            pl.BlockSpec(memory_space=pltpu.VMEM),
            pl.BlockSpec(memory_space=pltpu.VMEM),
        ],
        out_specs=pl.BlockSpec(memory_space=pltpu.VMEM),
        scratch_shapes=[
            pltpu.VMEM((N_DEV, m_per, k), jnp.float32),
            pltpu.SemaphoreType.DMA((H_R,)),
            pltpu.SemaphoreType.DMA((H_R,)),
            pltpu.SemaphoreType.DMA((H_L,)),
            pltpu.SemaphoreType.DMA((H_L,)),
        ],
        compiler_params=pltpu.CompilerParams(collective_id=0),
    )(x, w_mat)


# device time: 37657 ns/iter; 2.0681x vs baseline; 1.1607x over previous
import jax
import jax.numpy as jnp
from jax import lax
from jax.experimental import pallas as pl
from jax.experimental.pallas import tpu as pltpu

N_DEV = 16
H_R = 8
H_L = 7
SEG = 4


def kernel(x, w_mat):
    m_per, k = x.shape
    _, n_per = w_mat.shape
    seg_m = m_per // SEG

    def ls(d):
        return 0 if d == 0 else H_R + d

    def body(x_ref, w_ref, out_ref, comm_ref,
             send_r, recv_r, send_l, recv_l):
        my_pos = lax.axis_index("i")
        left = lax.rem(my_pos + N_DEV - 1, N_DEV)
        right = lax.rem(my_pos + 1, N_DEV)

        barrier_sem = pltpu.get_barrier_semaphore()
        for nbr in (left, right):
            pl.semaphore_signal(
                barrier_sem, inc=1,
                device_id=(nbr,), device_id_type=pl.DeviceIdType.MESH,
            )
        pl.semaphore_wait(barrier_sem, 2)

        comm_ref[0] = x_ref[...]

        def send(slot_src, slot_dst, ssem, rsem, h, s, target):
            rdma = pltpu.make_async_remote_copy(
                src_ref=comm_ref.at[slot_src, pl.ds(s * seg_m, seg_m)],
                dst_ref=comm_ref.at[slot_dst, pl.ds(s * seg_m, seg_m)],
                send_sem=ssem.at[h, s],
                recv_sem=rsem.at[h, s],
                device_id=(target,),
                device_id_type=pl.DeviceIdType.MESH,
            )
            rdma.start()
            return rdma

        def recv(slot_dst, rsem, h, s, src_dev):
            return pltpu.make_async_remote_copy(
                src_ref=comm_ref.at[slot_dst, pl.ds(s * seg_m, seg_m)],
                dst_ref=comm_ref.at[slot_dst, pl.ds(s * seg_m, seg_m)],
                send_sem=rsem.at[h, s],
                recv_sem=rsem.at[h, s],
                device_id=(src_dev,),
                device_id_type=pl.DeviceIdType.MESH,
            )

        def gemm(slot, origin):
            out_ref[pl.ds(origin * m_per, m_per), :] = jnp.dot(
                comm_ref[slot], w_ref[...],
                preferred_element_type=jnp.float32,
            )

        sends = []
        for s in range(SEG):
            sends.append(send(0, 1, send_r, recv_r, 0, s, right))
            sends.append(send(0, ls(1), send_l, recv_l, 0, s, left))
        gemm(0, my_pos)

        for h in range(H_R):
            do_l = h < H_L
            for s in range(SEG):
                recv(h + 1, recv_r, h, s, left).wait_recv()
                if h + 1 < H_R:
                    sends.append(
                        send(h + 1, h + 2, send_r, recv_r, h + 1, s, right)
                    )
                if do_l:
                    recv(ls(h + 1), recv_l, h, s, right).wait_recv()
                    if h + 1 < H_L:
                        sends.append(
                            send(ls(h + 1), ls(h + 2),
                                 send_l, recv_l, h + 1, s, left)
                        )
            gemm(h + 1, lax.rem(my_pos + N_DEV - (h + 1), N_DEV))
            if do_l:
                gemm(ls(h + 1), lax.rem(my_pos + h + 1, N_DEV))

        for sd in sends:
            sd.wait_send()

    return pl.pallas_call(
        body,
        out_shape=jax.ShapeDtypeStruct((N_DEV * m_per, n_per), jnp.float32),
        in_specs=[
            pl.BlockSpec(memory_space=pltpu.VMEM),
            pl.BlockSpec(memory_space=pltpu.VMEM),
        ],
        out_specs=pl.BlockSpec(memory_space=pltpu.VMEM),
        scratch_shapes=[
            pltpu.VMEM((N_DEV, m_per, k), jnp.float32),
            pltpu.SemaphoreType.DMA((H_R, SEG)),
            pltpu.SemaphoreType.DMA((H_R, SEG)),
            pltpu.SemaphoreType.DMA((H_L, SEG)),
            pltpu.SemaphoreType.DMA((H_L, SEG)),
        ],
        compiler_params=pltpu.CompilerParams(collective_id=0),
    )(x, w_mat)


# device time: 26392 ns/iter; 2.9509x vs baseline; 1.4268x over previous
import jax
import jax.numpy as jnp
from jax import lax
from jax.experimental import pallas as pl
from jax.experimental.pallas import tpu as pltpu

N_DEV = 16
DGS = (0, 1, 2, 3, -1, -2, -3)
SEGS = 2
HALF_SEGS = SEGS // 2


def kernel(x, w_mat):
    m_per, k = x.shape
    _, n_per = w_mat.shape
    seg_m = m_per // SEGS

    def slot(dj, dgi):
        return dj * 7 + dgi

    def body(x_ref, w_ref, out_ref, comm_ref,
             zu_s, zu_r, zd_s, zd_r,
             p1_s, p1_r,
             p2_s, p2_r):
        my_pos = lax.axis_index("i")
        j = lax.rem(my_pos, 4)
        g = my_pos // 4
        jr = g * 4 + lax.rem(j + 1, 4)
        jl = g * 4 + lax.rem(j + 3, 4)
        up = my_pos + 4
        dn = my_pos - 4

        def valid(dgi):
            dg = DGS[dgi]
            return jnp.logical_and(g + dg >= 0, g + dg <= 3)

        barrier_sem = pltpu.get_barrier_semaphore()
        for nbr in (jl, jr):
            pl.semaphore_signal(
                barrier_sem, inc=1,
                device_id=(nbr,), device_id_type=pl.DeviceIdType.MESH,
            )

        @pl.when(g < 3)
        def _():
            pl.semaphore_signal(
                barrier_sem, inc=1,
                device_id=(up,), device_id_type=pl.DeviceIdType.MESH,
            )

        @pl.when(g > 0)
        def _():
            pl.semaphore_signal(
                barrier_sem, inc=1,
                device_id=(dn,), device_id_type=pl.DeviceIdType.MESH,
            )

        @pl.when(g == 0)
        def _():
            pl.semaphore_signal(barrier_sem, inc=1)

        @pl.when(g == 3)
        def _():
            pl.semaphore_signal(barrier_sem, inc=1)

        pl.semaphore_wait(barrier_sem, 4)

        comm_ref[slot(0, 0)] = x_ref[...]

        def rdma(src_slot, dst_slot, ssem, rsem, target, row0, rows):
            return pltpu.make_async_remote_copy(
                src_ref=comm_ref.at[src_slot, pl.ds(row0, rows)],
                dst_ref=comm_ref.at[dst_slot, pl.ds(row0, rows)],
                send_sem=ssem,
                recv_sem=rsem,
                device_id=(target,),
                device_id_type=pl.DeviceIdType.MESH,
            )

        def gemm(sl, origin):
            out_ref[pl.ds(origin * m_per, m_per), :] = jnp.dot(
                comm_ref[sl], w_ref[...],
                preferred_element_type=jnp.float32,
            )

        def origin_of(dj, dgi):
            djj = (0, -1, 1, 2)[dj]
            return (g + DGS[dgi]) * 4 + lax.rem(j + djj + 4, 4)

        ZU_SRC = (0, 4, 5)
        ZD_SRC = (0, 1, 2)

        def z_send_up(kk, s):
            @pl.when(jnp.logical_and(g < 3, valid(ZU_SRC[kk])))
            def _():
                rdma(slot(0, ZU_SRC[kk]), slot(0, 4 + kk),
                     zu_s.at[kk, s], zu_r.at[kk, s], up,
                     s * seg_m, seg_m).start()

        def z_send_dn(kk, s):
            @pl.when(jnp.logical_and(g > 0, valid(ZD_SRC[kk])))
            def _():
                rdma(slot(0, ZD_SRC[kk]), slot(0, 1 + kk),
                     zd_s.at[kk, s], zd_r.at[kk, s], dn,
                     s * seg_m, seg_m).start()

        def plane_d1(dgi, s):
            esrc = slot(0, dgi)

            @pl.when(valid(dgi))
            def _():
                rdma(esrc, slot(1, dgi),
                     p1_s.at[0, dgi, s], p1_r.at[0, dgi, s], jr,
                     s * seg_m, seg_m).start()
                rdma(esrc, slot(2, dgi),
                     p1_s.at[1, dgi, s], p1_r.at[1, dgi, s], jl,
                     s * seg_m, seg_m).start()

        def recv_wait(dst_slot, rsem, row0, rows):
            rdma(dst_slot, dst_slot, rsem, rsem, my_pos,
                 row0, rows).wait_recv()

        for s in range(SEGS):
            z_send_up(0, s)
            z_send_dn(0, s)
            plane_d1(0, s)
        gemm(slot(0, 0), my_pos)

        for depth in range(1, 6):
            if depth <= 3:
                kk = depth - 1
                dgi_dn = 4 + kk
                dgi_up = 1 + kk

                for s in range(SEGS):
                    @pl.when(valid(dgi_dn))
                    def _(kk=kk, dgi_dn=dgi_dn, s=s):
                        recv_wait(slot(0, dgi_dn), zu_r.at[kk, s],
                                  s * seg_m, seg_m)

                    if depth < 3:
                        z_send_up(kk + 1, s)
                    plane_d1(dgi_dn, s)

                    @pl.when(valid(dgi_up))
                    def _(kk=kk, dgi_up=dgi_up, s=s):
                        recv_wait(slot(0, dgi_up), zd_r.at[kk, s],
                                  s * seg_m, seg_m)

                    if depth < 3:
                        z_send_dn(kk + 1, s)
                    plane_d1(dgi_up, s)

                @pl.when(valid(dgi_dn))
                def _(dgi_dn=dgi_dn):
                    gemm(slot(0, dgi_dn), origin_of(0, dgi_dn))

                @pl.when(valid(dgi_up))
                def _(dgi_up=dgi_up):
                    gemm(slot(0, dgi_up), origin_of(0, dgi_up))

            if 1 <= depth <= 4:
                d1s = [0] if depth == 1 else [4 + depth - 2, 1 + depth - 2]
                for dgi in d1s:
                    @pl.when(valid(dgi))
                    def _(dgi=dgi):
                        for s in range(SEGS):
                            rs = s
                            recv_wait(slot(1, dgi), p1_r.at[0, dgi, s],
                                      s * seg_m, seg_m)
                            if s < HALF_SEGS:
                                rdma(slot(1, dgi), slot(3, dgi),
                                     p2_s.at[0, dgi, s],
                                     p2_r.at[0, dgi, s], jr,
                                     s * seg_m, seg_m).start()
                            recv_wait(slot(2, dgi), p1_r.at[1, dgi, rs],
                                      rs * seg_m, seg_m)
                            if rs >= HALF_SEGS:
                                rdma(slot(2, dgi), slot(3, dgi),
                                     p2_s.at[1, dgi, rs - HALF_SEGS],
                                     p2_r.at[1, dgi, rs - HALF_SEGS], jl,
                                     rs * seg_m, seg_m).start()
                        gemm(slot(1, dgi), origin_of(1, dgi))
                        gemm(slot(2, dgi), origin_of(2, dgi))

            if depth >= 2:
                d2s = [0] if depth == 2 else [4 + depth - 3, 1 + depth - 3]
                for dgi in d2s:
                    @pl.when(valid(dgi))
                    def _(dgi=dgi):
                        for q in range(HALF_SEGS):
                            recv_wait(slot(3, dgi), p2_r.at[0, dgi, q],
                                      q * seg_m, seg_m)
                            recv_wait(slot(3, dgi), p2_r.at[1, dgi, q],
                                      (HALF_SEGS + q) * seg_m, seg_m)
                        gemm(slot(3, dgi), origin_of(3, dgi))

        for kk in range(3):
            for s in range(SEGS):
                @pl.when(jnp.logical_and(g < 3, valid(ZU_SRC[kk])))
                def _(kk=kk, s=s):
                    rdma(slot(0, ZU_SRC[kk]), slot(0, 4 + kk),
                         zu_s.at[kk, s], zu_r.at[kk, s], up,
                         s * seg_m, seg_m).wait_send()

                @pl.when(jnp.logical_and(g > 0, valid(ZD_SRC[kk])))
                def _(kk=kk, s=s):
                    rdma(slot(0, ZD_SRC[kk]), slot(0, 1 + kk),
                         zd_s.at[kk, s], zd_r.at[kk, s], dn,
                         s * seg_m, seg_m).wait_send()

        for dgi in range(7):
            @pl.when(valid(dgi))
            def _(dgi=dgi):
                esrc = slot(0, dgi)
                for s in range(SEGS):
                    rdma(esrc, slot(1, dgi), p1_s.at[0, dgi, s],
                         p1_r.at[0, dgi, s], jr,
                         s * seg_m, seg_m).wait_send()
                    rdma(esrc, slot(2, dgi), p1_s.at[1, dgi, s],
                         p1_r.at[1, dgi, s], jl,
                         s * seg_m, seg_m).wait_send()
                for q in range(HALF_SEGS):
                    rdma(slot(1, dgi), slot(3, dgi), p2_s.at[0, dgi, q],
                         p2_r.at[0, dgi, q], jr,
                         q * seg_m, seg_m).wait_send()
                    rdma(slot(2, dgi), slot(3, dgi), p2_s.at[1, dgi, q],
                         p2_r.at[1, dgi, q], jl,
                         (HALF_SEGS + q) * seg_m, seg_m).wait_send()

    return pl.pallas_call(
        body,
        out_shape=jax.ShapeDtypeStruct((N_DEV * m_per, n_per), jnp.float32),
        in_specs=[
            pl.BlockSpec(memory_space=pltpu.VMEM),
            pl.BlockSpec(memory_space=pltpu.VMEM),
        ],
        out_specs=pl.BlockSpec(memory_space=pltpu.VMEM),
        scratch_shapes=[
            pltpu.VMEM((28, m_per, k), jnp.float32),
            pltpu.SemaphoreType.DMA((3, SEGS)),
            pltpu.SemaphoreType.DMA((3, SEGS)),
            pltpu.SemaphoreType.DMA((3, SEGS)),
            pltpu.SemaphoreType.DMA((3, SEGS)),
            pltpu.SemaphoreType.DMA((2, 7, SEGS)),
            pltpu.SemaphoreType.DMA((2, 7, SEGS)),
            pltpu.SemaphoreType.DMA((2, 7, HALF_SEGS)),
            pltpu.SemaphoreType.DMA((2, 7, HALF_SEGS)),
        ],
        compiler_params=pltpu.CompilerParams(collective_id=0),
    )(x, w_mat)
